# baseline (speedup 1.0000x reference)
import functools

import jax
import jax.numpy as jnp
from jax import lax
from jax.experimental import pallas as pl
from jax.experimental.pallas import tpu as pltpu
from jax.experimental.pallas import tpu_sc as plsc

NUM_EXPERTS = 8
TOP_K = 2
LANES = 16
NUM_WORKERS = 32
TBLK = 2048


def _gate_logits_body(w_ref, x_ref, out_ref):
    out_ref[...] = lax.dot_general(
        w_ref[...], x_ref[...],
        dimension_numbers=(((1,), (1,)), ((), ())),
        preferred_element_type=jnp.float32)


def _gate_logits(x, w, tokens):
    hidden = x.shape[1]
    return pl.pallas_call(
        _gate_logits_body,
        grid=(tokens // TBLK,),
        in_specs=[
            pl.BlockSpec((NUM_EXPERTS, hidden), lambda i: (0, 0)),
            pl.BlockSpec((TBLK, hidden), lambda i: (i, 0)),
        ],
        out_specs=pl.BlockSpec((NUM_EXPERTS, TBLK), lambda i: (0, i)),
        out_shape=jax.ShapeDtypeStruct((NUM_EXPERTS, tokens), jnp.float32),
        compiler_params=pltpu.CompilerParams(
            dimension_semantics=("arbitrary",)),
    )(w, x)


def _routing_body(tok_per_w, tokens, logits_hbm, w_hbm, e_hbm, lv, wv, ev):
    wid = lax.axis_index("s") * 2 + lax.axis_index("c")
    base = wid * tok_per_w
    for e in range(NUM_EXPERTS):
        pltpu.sync_copy(logits_hbm.at[pl.ds(e * tokens + base, tok_per_w)],
                        lv.at[pl.ds(e * tok_per_w, tok_per_w)])

    neg_inf = jnp.full((LANES,), -jnp.inf, jnp.float32)

    def group(g, carry):
        t0 = g * LANES
        ls = [lv[pl.ds(e * tok_per_w + t0, LANES)] for e in range(NUM_EXPERTS)]
        m1 = ls[0]
        for e in range(1, NUM_EXPERTS):
            m1 = jnp.maximum(m1, ls[e])
        e1 = jnp.zeros((LANES,), jnp.int32)
        for e in range(NUM_EXPERTS - 1, -1, -1):
            e1 = jnp.where(ls[e] == m1, jnp.full((LANES,), e, jnp.int32), e1)
        ls2 = [jnp.where(e1 == jnp.full((LANES,), e, jnp.int32), neg_inf, ls[e])
               for e in range(NUM_EXPERTS)]
        m2 = ls2[0]
        for e in range(1, NUM_EXPERTS):
            m2 = jnp.maximum(m2, ls2[e])
        e2 = jnp.zeros((LANES,), jnp.int32)
        for e in range(NUM_EXPERTS - 1, -1, -1):
            e2 = jnp.where(ls2[e] == m2, jnp.full((LANES,), e, jnp.int32), e2)
        t = jnp.exp(m2 - m1)
        w1 = 1.0 / (1.0 + t)
        w2 = t * w1
        wv[pl.ds(t0, LANES)] = w1
        wv[pl.ds(tok_per_w + t0, LANES)] = w2
        ev[pl.ds(t0, LANES)] = e1
        ev[pl.ds(tok_per_w + t0, LANES)] = e2
        return carry

    lax.fori_loop(0, tok_per_w // LANES, group, 0)
    for k in range(TOP_K):
        pltpu.sync_copy(wv.at[pl.ds(k * tok_per_w, tok_per_w)],
                        w_hbm.at[pl.ds(k * tokens + base, tok_per_w)])
        pltpu.sync_copy(ev.at[pl.ds(k * tok_per_w, tok_per_w)],
                        e_hbm.at[pl.ds(k * tokens + base, tok_per_w)])


def _routing(logits_flat, tokens):
    tok_per_w = tokens // NUM_WORKERS
    mesh = plsc.VectorSubcoreMesh(core_axis_name="c", subcore_axis_name="s")
    fn = pl.kernel(
        functools.partial(_routing_body, tok_per_w, tokens),
        mesh=mesh,
        out_type=[
            jax.ShapeDtypeStruct((TOP_K * tokens,), jnp.float32),
            jax.ShapeDtypeStruct((TOP_K * tokens,), jnp.int32),
        ],
        scratch_types=[
            pltpu.VMEM((NUM_EXPERTS * tok_per_w,), jnp.float32),
            pltpu.VMEM((TOP_K * tok_per_w,), jnp.float32),
            pltpu.VMEM((TOP_K * tok_per_w,), jnp.int32),
        ],
    )
    return fn(logits_flat)


def kernel(hidden_states, gate_weight):
    batch, seq, hidden = hidden_states.shape
    tokens = batch * seq
    x = hidden_states.reshape(tokens, hidden)
    logits_t = _gate_logits(x, gate_weight, tokens)
    w_flat, e_flat = _routing(logits_t.reshape(-1), tokens)
    w = w_flat.reshape(TOP_K, tokens).T
    e = e_flat.reshape(TOP_K, tokens).T
    return (w, e)


# speedup vs baseline: 1.0116x; 1.0116x over previous
import functools

import jax
import jax.numpy as jnp
from jax import lax
from jax.experimental import pallas as pl
from jax.experimental.pallas import tpu as pltpu
from jax.experimental.pallas import tpu_sc as plsc

NUM_EXPERTS = 8
TOP_K = 2
LANES = 16
NUM_WORKERS = 32
TBLK = 1024


def _gate_logits_body(w_ref, x_ref, out_ref):
    out_ref[...] = lax.dot_general(
        w_ref[...], x_ref[...],
        dimension_numbers=(((1,), (1,)), ((), ())),
        preferred_element_type=jnp.float32)


def _gate_logits(x, w, tokens):
    hidden = x.shape[1]
    return pl.pallas_call(
        _gate_logits_body,
        grid=(tokens // TBLK,),
        in_specs=[
            pl.BlockSpec((NUM_EXPERTS, hidden), lambda i: (0, 0)),
            pl.BlockSpec((TBLK, hidden), lambda i: (i, 0)),
        ],
        out_specs=pl.BlockSpec((NUM_EXPERTS, TBLK), lambda i: (0, i)),
        out_shape=jax.ShapeDtypeStruct((NUM_EXPERTS, tokens), jnp.float32),
        compiler_params=pltpu.CompilerParams(
            dimension_semantics=("arbitrary",)),
    )(w, x)


def _routing_body(tok_per_w, tokens, logits_hbm, w_hbm, e_hbm, lv, wv, ev):
    wid = lax.axis_index("s") * 2 + lax.axis_index("c")
    base = wid * tok_per_w
    for e in range(NUM_EXPERTS):
        pltpu.sync_copy(logits_hbm.at[pl.ds(e * tokens + base, tok_per_w)],
                        lv.at[pl.ds(e * tok_per_w, tok_per_w)])

    neg_inf = jnp.full((LANES,), -jnp.inf, jnp.float32)

    def group(g, carry):
        t0 = g * LANES
        ls = [lv[pl.ds(e * tok_per_w + t0, LANES)] for e in range(NUM_EXPERTS)]
        m1 = ls[0]
        for e in range(1, NUM_EXPERTS):
            m1 = jnp.maximum(m1, ls[e])
        e1 = jnp.zeros((LANES,), jnp.int32)
        for e in range(NUM_EXPERTS - 1, -1, -1):
            e1 = jnp.where(ls[e] == m1, jnp.full((LANES,), e, jnp.int32), e1)
        ls2 = [jnp.where(e1 == jnp.full((LANES,), e, jnp.int32), neg_inf, ls[e])
               for e in range(NUM_EXPERTS)]
        m2 = ls2[0]
        for e in range(1, NUM_EXPERTS):
            m2 = jnp.maximum(m2, ls2[e])
        e2 = jnp.zeros((LANES,), jnp.int32)
        for e in range(NUM_EXPERTS - 1, -1, -1):
            e2 = jnp.where(ls2[e] == m2, jnp.full((LANES,), e, jnp.int32), e2)
        t = jnp.exp(m2 - m1)
        w1 = 1.0 / (1.0 + t)
        w2 = t * w1
        wv[pl.ds(t0, LANES)] = w1
        wv[pl.ds(tok_per_w + t0, LANES)] = w2
        ev[pl.ds(t0, LANES)] = e1
        ev[pl.ds(tok_per_w + t0, LANES)] = e2
        return carry

    lax.fori_loop(0, tok_per_w // LANES, group, 0)
    for k in range(TOP_K):
        pltpu.sync_copy(wv.at[pl.ds(k * tok_per_w, tok_per_w)],
                        w_hbm.at[pl.ds(k * tokens + base, tok_per_w)])
        pltpu.sync_copy(ev.at[pl.ds(k * tok_per_w, tok_per_w)],
                        e_hbm.at[pl.ds(k * tokens + base, tok_per_w)])


def _routing(logits_flat, tokens):
    tok_per_w = tokens // NUM_WORKERS
    mesh = plsc.VectorSubcoreMesh(core_axis_name="c", subcore_axis_name="s")
    fn = pl.kernel(
        functools.partial(_routing_body, tok_per_w, tokens),
        mesh=mesh,
        out_type=[
            jax.ShapeDtypeStruct((TOP_K * tokens,), jnp.float32),
            jax.ShapeDtypeStruct((TOP_K * tokens,), jnp.int32),
        ],
        scratch_types=[
            pltpu.VMEM((NUM_EXPERTS * tok_per_w,), jnp.float32),
            pltpu.VMEM((TOP_K * tok_per_w,), jnp.float32),
            pltpu.VMEM((TOP_K * tok_per_w,), jnp.int32),
        ],
    )
    return fn(logits_flat)


def kernel(hidden_states, gate_weight):
    batch, seq, hidden = hidden_states.shape
    tokens = batch * seq
    x = hidden_states.reshape(tokens, hidden)
    logits_t = _gate_logits(x, gate_weight, tokens)
    w_flat, e_flat = _routing(logits_t.reshape(-1), tokens)
    w = w_flat.reshape(TOP_K, tokens).T
    e = e_flat.reshape(TOP_K, tokens).T
    return (w, e)
